# baseline (speedup 1.0000x reference)
import functools

import jax
import jax.numpy as jnp
from jax.experimental import pallas as pl
from jax.experimental.pallas import tpu as pltpu

_H, _W = 512, 512
_N = 100
_MH, _MW = 28, 28
_WIN = 224
_CWIN = 384
_MASK_BIN = 0.5
_SCORE_T = 0.5
_OVERLAP_T = 0.5
_STUFF_AREA = 4096.0
_OFFSET = 90
_NUM_SEM = 20


def _panoptic_kernel(sb_ref, masks_ref, seg_ref, cat_ref, inst_ref):
    f32 = jnp.float32
    bf16 = jnp.bfloat16
    cat_ref[...] = jnp.zeros((_H, _W), f32)
    inst_ref[...] = jnp.full((_H, _W), -1.0, f32)

    cwin_iota = jax.lax.broadcasted_iota(jnp.int32, (1, _CWIN), 1)
    mcol = jax.lax.broadcasted_iota(jnp.int32, (_MW, _CWIN), 0)
    mrow = jax.lax.broadcasted_iota(jnp.int32, (_WIN, _MH), 1)
    win_iota = jax.lax.broadcasted_iota(jnp.int32, (_WIN, 1), 0).astype(f32)
    ones_row = jnp.ones((1, _WIN), bf16)

    def cond(i):
        s = sb_ref[4, jnp.minimum(i, _N - 1)]
        return (i < _N) & (s > _SCORE_T)

    def body(i):
        y1 = sb_ref[0, i]
        x1 = sb_ref[1, i]
        y2 = sb_ref[2, i]
        x2 = sb_ref[3, i]
        cls = sb_ref[5, i]
        idx = sb_ref[6, i].astype(jnp.int32)
        h = jnp.maximum(y2 - y1, 1e-4)
        w = jnp.maximum(x2 - x1, 1e-4)

        chalf = jnp.minimum(jnp.maximum(x1.astype(jnp.int32) // 128, 0), 1)
        xs = (chalf * 128 + cwin_iota).astype(f32) + 0.5
        u = (xs - x1) / w * _MW - 0.5
        u0 = jnp.floor(u)
        wu = u - u0
        u0i = u0.astype(jnp.int32)
        inside_x = ((xs >= x1) & (xs < x2)).astype(f32)
        wut = (jnp.where(mcol == u0i, 1.0 - wu, 0.0)
               + jnp.where(mcol == u0i + 1, wu, 0.0)) * inside_x

        r0 = jnp.minimum(jnp.maximum(y1.astype(jnp.int32) // 8, 0),
                         (_H - _WIN) // 8) * 8
        ys = r0.astype(f32) + win_iota + 0.5
        v = (ys - y1) / h * _MH - 0.5
        v0 = jnp.floor(v)
        wv = v - v0
        v0i = v0.astype(jnp.int32)
        inside_y = ((ys >= y1) & (ys < y2)).astype(f32)
        wvm = (jnp.where(mrow == v0i, 1.0 - wv, 0.0)
               + jnp.where(mrow == v0i + 1, wv, 0.0)) * inside_y

        mask = masks_ref[idx]
        tmp = jax.lax.dot_general(
            mask, wut, (((1,), (0,)), ((), ())),
            precision=jax.lax.Precision.HIGHEST,
            preferred_element_type=f32)
        pm = jax.lax.dot_general(
            wvm, tmp, (((1,), (0,)), ((), ())),
            precision=jax.lax.Precision.HIGHEST,
            preferred_element_type=f32)

        binm = pm > _MASK_BIN
        binb = jnp.where(binm, 1.0, 0.0).astype(bf16)
        area = jnp.sum(jax.lax.dot_general(
            ones_row, binb, (((1,), (0,)), ((), ())),
            preferred_element_type=f32))

        def half(lo):
            cat_win = cat_ref[pl.ds(r0, _WIN), lo:lo + _CWIN]
            claimed = cat_win != 0.0
            ovb = jnp.where(binm & claimed, 1.0, 0.0).astype(bf16)
            ov = jnp.sum(jax.lax.dot_general(
                ones_row, ovb, (((1,), (0,)), ((), ())),
                preferred_element_type=f32))
            ok = (area > 0.0) & (ov / jnp.maximum(area, 1.0) <= _OVERLAP_T)

            @pl.when(ok)
            def _():
                new = binm & jnp.logical_not(claimed)
                cat_ref[pl.ds(r0, _WIN), lo:lo + _CWIN] = jnp.where(
                    new, cls, cat_win)
                inst_win = inst_ref[pl.ds(r0, _WIN), lo:lo + _CWIN]
                inst_ref[pl.ds(r0, _WIN), lo:lo + _CWIN] = jnp.where(
                    new, (idx + 1).astype(f32), inst_win)

        pl.when(chalf == 0)(lambda: half(0))
        pl.when(chalf == 1)(lambda: half(128))
        return i + 1

    jax.lax.while_loop(cond, body, 0)

    seg = seg_ref[...]
    cat = cat_ref[...]
    free = cat == 0.0
    freeseg = jnp.where(free, seg, 31)
    kbits = jnp.int32(0)
    for s in range(2, _NUM_SEM):
        cnt = jnp.sum((freeseg == s).astype(f32))
        kbits += jnp.where(cnt >= _STUFF_AREA, jnp.int32(1 << s), 0)
    hit = jnp.bitwise_and(jnp.right_shift(kbits, freeseg), 1) == 1
    cat_ref[...] = jnp.where(hit, seg.astype(f32) + float(_OFFSET), cat)


@functools.partial(jax.jit, static_argnums=())
def _run_single(boxes, scores, classes, masks, seg):
    order = jnp.argsort(-scores)
    bx = boxes[order]
    sb = jnp.stack([
        bx[:, 0], bx[:, 1], bx[:, 2], bx[:, 3],
        scores[order], classes[order].astype(jnp.float32),
        order.astype(jnp.float32), jnp.zeros((_N,), jnp.float32),
    ], axis=0)
    sb = jnp.pad(sb, ((0, 0), (0, 128 - _N)))

    cat, inst = pl.pallas_call(
        _panoptic_kernel,
        out_shape=(
            jax.ShapeDtypeStruct((_H, _W), jnp.float32),
            jax.ShapeDtypeStruct((_H, _W), jnp.float32),
        ),
        in_specs=[
            pl.BlockSpec(memory_space=pltpu.SMEM),
            pl.BlockSpec(memory_space=pltpu.VMEM),
            pl.BlockSpec(memory_space=pltpu.VMEM),
        ],
        out_specs=(
            pl.BlockSpec(memory_space=pltpu.VMEM),
            pl.BlockSpec(memory_space=pltpu.VMEM),
        ),
    )(sb, masks, seg)
    return cat, inst


def kernel(boxes, scores, classes, masks, segmentation_mask):
    B = boxes.shape[0]
    cats, insts = [], []
    for b in range(B):
        c, i = _run_single(boxes[b], scores[b], classes[b],
                           masks[b, :, :, :, 0], segmentation_mask[b])
        cats.append(c)
        insts.append(i)
    return jnp.stack(cats), jnp.stack(insts)


# speedup vs baseline: 1.1363x; 1.1363x over previous
import functools

import jax
import jax.numpy as jnp
from jax.experimental import pallas as pl
from jax.experimental.pallas import tpu as pltpu

_H, _W = 512, 512
_N = 100
_MH, _MW = 28, 28
_WIN = 224
_CWIN = 384
_MASK_BIN = 0.5
_SCORE_T = 0.5
_OVERLAP_T = 0.5
_STUFF_AREA = 4096.0
_OFFSET = 90
_NUM_SEM = 20


def _panoptic_kernel(sb_ref, masks_ref, seg_ref, cat_ref, inst_ref,
                     binf_ref, area_ref):
    f32 = jnp.float32
    cat_ref[...] = jnp.zeros((_H, _W), f32)
    inst_ref[...] = jnp.full((_H, _W), -1.0, f32)

    cwin_iota = jax.lax.broadcasted_iota(jnp.int32, (1, _CWIN), 1)
    mcol = jax.lax.broadcasted_iota(jnp.int32, (_MW, _CWIN), 0)
    mrow = jax.lax.broadcasted_iota(jnp.int32, (_WIN, _MH), 1)
    win_iota = jax.lax.broadcasted_iota(jnp.int32, (_WIN, 1), 0).astype(f32)

    def window(j):
        y1 = sb_ref[0, j]
        x1 = sb_ref[1, j]
        r0 = jnp.minimum(jnp.maximum(y1.astype(jnp.int32) // 8, 0),
                         (_H - _WIN) // 8) * 8
        chalf = jnp.minimum(jnp.maximum(x1.astype(jnp.int32) // 128, 0), 1)
        return r0, chalf

    def paste(j):
        y1 = sb_ref[0, j]
        x1 = sb_ref[1, j]
        y2 = sb_ref[2, j]
        x2 = sb_ref[3, j]
        idx = sb_ref[6, j].astype(jnp.int32)
        h = jnp.maximum(y2 - y1, 1e-4)
        w = jnp.maximum(x2 - x1, 1e-4)
        r0, chalf = window(j)

        xs = (chalf * 128 + cwin_iota).astype(f32) + 0.5
        u = (xs - x1) / w * _MW - 0.5
        u0 = jnp.floor(u)
        wu = u - u0
        u0i = u0.astype(jnp.int32)
        inside_x = ((xs >= x1) & (xs < x2)).astype(f32)
        wut = (jnp.where(mcol == u0i, 1.0 - wu, 0.0)
               + jnp.where(mcol == u0i + 1, wu, 0.0)) * inside_x

        ys = r0.astype(f32) + win_iota + 0.5
        v = (ys - y1) / h * _MH - 0.5
        v0 = jnp.floor(v)
        wv = v - v0
        v0i = v0.astype(jnp.int32)
        inside_y = ((ys >= y1) & (ys < y2)).astype(f32)
        wvm = (jnp.where(mrow == v0i, 1.0 - wv, 0.0)
               + jnp.where(mrow == v0i + 1, wv, 0.0)) * inside_y

        mask = masks_ref[idx]
        tmp = jax.lax.dot_general(
            mask, wut, (((1,), (0,)), ((), ())),
            precision=jax.lax.Precision.HIGHEST,
            preferred_element_type=f32)
        pm = jax.lax.dot_general(
            wvm, tmp, (((1,), (0,)), ((), ())),
            precision=jax.lax.Precision.HIGHEST,
            preferred_element_type=f32)

        binf = jnp.where(pm > _MASK_BIN, 1.0, 0.0)
        binf_ref[...] = binf
        area_ref[0] = jnp.sum(binf)

    def cond(i):
        s = sb_ref[4, jnp.minimum(i, _N - 1)]
        return (i < _N) & (s > _SCORE_T)

    def body(i):
        cls = sb_ref[5, i]
        idx = sb_ref[6, i].astype(jnp.int32)
        r0, chalf = window(i)
        area = area_ref[0]
        binf = binf_ref[...]

        def half(lo):
            cat_win = cat_ref[pl.ds(r0, _WIN), lo:lo + _CWIN]
            claimed = cat_win != 0.0
            ov = jnp.sum(jnp.where(claimed, binf, 0.0))
            ok = (area > 0.0) & (ov / jnp.maximum(area, 1.0) <= _OVERLAP_T)

            @pl.when(ok)
            def _():
                new = (binf != 0.0) & jnp.logical_not(claimed)
                cat_ref[pl.ds(r0, _WIN), lo:lo + _CWIN] = jnp.where(
                    new, cls, cat_win)
                inst_win = inst_ref[pl.ds(r0, _WIN), lo:lo + _CWIN]
                inst_ref[pl.ds(r0, _WIN), lo:lo + _CWIN] = jnp.where(
                    new, (idx + 1).astype(f32), inst_win)

        pl.when(chalf == 0)(lambda: half(0))
        pl.when(chalf == 1)(lambda: half(128))

        j = i + 1
        nextok = (j < _N) & (sb_ref[4, jnp.minimum(j, _N - 1)] > _SCORE_T)
        pl.when(nextok)(lambda: paste(j))
        return i + 1

    pl.when(sb_ref[4, 0] > _SCORE_T)(lambda: paste(0))
    jax.lax.while_loop(cond, body, 0)

    seg = seg_ref[...]
    cat = cat_ref[...]
    free = cat == 0.0
    freeseg = jnp.where(free, seg, 31)
    kbits = jnp.int32(0)
    for s in range(2, _NUM_SEM):
        cnt = jnp.sum((freeseg == s).astype(f32))
        kbits += jnp.where(cnt >= _STUFF_AREA, jnp.int32(1 << s), 0)
    hit = jnp.bitwise_and(jnp.right_shift(kbits, freeseg), 1) == 1
    cat_ref[...] = jnp.where(hit, seg.astype(f32) + float(_OFFSET), cat)


@functools.partial(jax.jit, static_argnums=())
def _run_single(boxes, scores, classes, masks, seg):
    order = jnp.argsort(-scores)
    bx = boxes[order]
    sb = jnp.stack([
        bx[:, 0], bx[:, 1], bx[:, 2], bx[:, 3],
        scores[order], classes[order].astype(jnp.float32),
        order.astype(jnp.float32), jnp.zeros((_N,), jnp.float32),
    ], axis=0)
    sb = jnp.pad(sb, ((0, 0), (0, 128 - _N)))

    cat, inst = pl.pallas_call(
        _panoptic_kernel,
        out_shape=(
            jax.ShapeDtypeStruct((_H, _W), jnp.float32),
            jax.ShapeDtypeStruct((_H, _W), jnp.float32),
        ),
        in_specs=[
            pl.BlockSpec(memory_space=pltpu.SMEM),
            pl.BlockSpec(memory_space=pltpu.VMEM),
            pl.BlockSpec(memory_space=pltpu.VMEM),
        ],
        out_specs=(
            pl.BlockSpec(memory_space=pltpu.VMEM),
            pl.BlockSpec(memory_space=pltpu.VMEM),
        ),
        scratch_shapes=[
            pltpu.VMEM((_WIN, _CWIN), jnp.float32),
            pltpu.SMEM((1,), jnp.float32),
        ],
    )(sb, masks, seg)
    return cat, inst


def kernel(boxes, scores, classes, masks, segmentation_mask):
    B = boxes.shape[0]
    cats, insts = [], []
    for b in range(B):
        c, i = _run_single(boxes[b], scores[b], classes[b],
                           masks[b, :, :, :, 0], segmentation_mask[b])
        cats.append(c)
        insts.append(i)
    return jnp.stack(cats), jnp.stack(insts)


# speedup vs baseline: 1.1868x; 1.0444x over previous
import functools

import jax
import jax.numpy as jnp
from jax.experimental import pallas as pl
from jax.experimental.pallas import tpu as pltpu

_H, _W = 512, 512
_N = 100
_MH, _MW = 28, 28
_WIN = 224
_MASK_BIN = 0.5
_SCORE_T = 0.5
_OVERLAP_T = 0.5
_STUFF_AREA = 4096.0
_OFFSET = 90
_NUM_SEM = 20


def _panoptic_kernel(sb_ref, masks_ref, seg_ref, cat_ref, inst_ref,
                     binf_ref, area_ref):
    f32 = jnp.float32
    cat_ref[...] = jnp.zeros((_H, _W), f32)
    inst_ref[...] = jnp.full((_H, _W), -1.0, f32)

    cw_iota = jax.lax.broadcasted_iota(jnp.int32, (1, _W), 1)
    mcol = jax.lax.broadcasted_iota(jnp.int32, (_MW, _W), 0)
    mrow = jax.lax.broadcasted_iota(jnp.int32, (_WIN, _MH), 1)
    win_iota = jax.lax.broadcasted_iota(jnp.int32, (_WIN, 1), 0).astype(f32)

    def rowstart(j):
        y1 = sb_ref[0, j]
        return jnp.minimum(jnp.maximum(y1.astype(jnp.int32) // 8, 0),
                           (_H - _WIN) // 8) * 8

    def paste(j):
        y1 = sb_ref[0, j]
        x1 = sb_ref[1, j]
        y2 = sb_ref[2, j]
        x2 = sb_ref[3, j]
        idx = sb_ref[6, j].astype(jnp.int32)
        h = jnp.maximum(y2 - y1, 1e-4)
        w = jnp.maximum(x2 - x1, 1e-4)
        r0 = rowstart(j)

        xs = cw_iota.astype(f32) + 0.5
        u = (xs - x1) / w * _MW - 0.5
        u0 = jnp.floor(u)
        wu = u - u0
        u0i = u0.astype(jnp.int32)
        inside_x = ((xs >= x1) & (xs < x2)).astype(f32)
        wut = (jnp.where(mcol == u0i, 1.0 - wu, 0.0)
               + jnp.where(mcol == u0i + 1, wu, 0.0)) * inside_x

        ys = r0.astype(f32) + win_iota + 0.5
        v = (ys - y1) / h * _MH - 0.5
        v0 = jnp.floor(v)
        wv = v - v0
        v0i = v0.astype(jnp.int32)
        inside_y = ((ys >= y1) & (ys < y2)).astype(f32)
        wvm = (jnp.where(mrow == v0i, 1.0 - wv, 0.0)
               + jnp.where(mrow == v0i + 1, wv, 0.0)) * inside_y

        mask = masks_ref[idx]
        tmp = jax.lax.dot_general(
            mask, wut, (((1,), (0,)), ((), ())),
            precision=jax.lax.Precision.HIGHEST,
            preferred_element_type=f32)
        pm = jax.lax.dot_general(
            wvm, tmp, (((1,), (0,)), ((), ())),
            precision=jax.lax.Precision.HIGHEST,
            preferred_element_type=f32)

        binf = jnp.where(pm > _MASK_BIN, 1.0, 0.0)
        binf_ref[...] = binf
        area_ref[0] = jnp.sum(binf)

    def cond(i):
        s = sb_ref[4, jnp.minimum(i, _N - 1)]
        return (i < _N) & (s > _SCORE_T)

    def body(i):
        cls = sb_ref[5, i]
        idx = sb_ref[6, i].astype(jnp.int32)
        r0 = rowstart(i)
        area = area_ref[0]
        binf = binf_ref[...]

        cat_win = cat_ref[pl.ds(r0, _WIN), :]
        claimed = cat_win != 0.0
        ov = jnp.sum(jnp.where(claimed, binf, 0.0))
        ok = (area > 0.0) & (ov / jnp.maximum(area, 1.0) <= _OVERLAP_T)
        new = (binf != 0.0) & jnp.logical_not(claimed) & ok
        cat_ref[pl.ds(r0, _WIN), :] = jnp.where(new, cls, cat_win)
        inst_win = inst_ref[pl.ds(r0, _WIN), :]
        inst_ref[pl.ds(r0, _WIN), :] = jnp.where(
            new, (idx + 1).astype(f32), inst_win)

        paste(jnp.minimum(i + 1, _N - 1))
        return i + 1

    pl.when(sb_ref[4, 0] > _SCORE_T)(lambda: paste(0))
    jax.lax.while_loop(cond, body, 0)

    seg = seg_ref[...]
    cat = cat_ref[...]
    free = cat == 0.0
    freeseg = jnp.where(free, seg, 31)
    kbits = jnp.int32(0)
    for s in range(2, _NUM_SEM):
        cnt = jnp.sum((freeseg == s).astype(f32))
        kbits += jnp.where(cnt >= _STUFF_AREA, jnp.int32(1 << s), 0)
    hit = jnp.bitwise_and(jnp.right_shift(kbits, freeseg), 1) == 1
    cat_ref[...] = jnp.where(hit, seg.astype(f32) + float(_OFFSET), cat)


@functools.partial(jax.jit, static_argnums=())
def _run_single(boxes, scores, classes, masks, seg):
    order = jnp.argsort(-scores)
    bx = boxes[order]
    sb = jnp.stack([
        bx[:, 0], bx[:, 1], bx[:, 2], bx[:, 3],
        scores[order], classes[order].astype(jnp.float32),
        order.astype(jnp.float32), jnp.zeros((_N,), jnp.float32),
    ], axis=0)
    sb = jnp.pad(sb, ((0, 0), (0, 128 - _N)))

    cat, inst = pl.pallas_call(
        _panoptic_kernel,
        out_shape=(
            jax.ShapeDtypeStruct((_H, _W), jnp.float32),
            jax.ShapeDtypeStruct((_H, _W), jnp.float32),
        ),
        in_specs=[
            pl.BlockSpec(memory_space=pltpu.SMEM),
            pl.BlockSpec(memory_space=pltpu.VMEM),
            pl.BlockSpec(memory_space=pltpu.VMEM),
        ],
        out_specs=(
            pl.BlockSpec(memory_space=pltpu.VMEM),
            pl.BlockSpec(memory_space=pltpu.VMEM),
        ),
        scratch_shapes=[
            pltpu.VMEM((_WIN, _W), jnp.float32),
            pltpu.SMEM((1,), jnp.float32),
        ],
    )(sb, masks, seg)
    return cat, inst


def kernel(boxes, scores, classes, masks, segmentation_mask):
    B = boxes.shape[0]
    cats, insts = [], []
    for b in range(B):
        c, i = _run_single(boxes[b], scores[b], classes[b],
                           masks[b, :, :, :, 0], segmentation_mask[b])
        cats.append(c)
        insts.append(i)
    return jnp.stack(cats), jnp.stack(insts)


# speedup vs baseline: 1.2754x; 1.0747x over previous
import functools

import jax
import jax.numpy as jnp
from jax.experimental import pallas as pl
from jax.experimental.pallas import tpu as pltpu

_H, _W = 512, 512
_N = 100
_MH, _MW = 28, 28
_WIN = 224
_CWIN = 384
_MASK_BIN = 0.5
_SCORE_T = 0.5
_OVERLAP_T = 0.5
_STUFF_AREA = 4096.0
_OFFSET = 90
_NUM_SEM = 20


def _panoptic_kernel(sb_ref, masks_ref, seg3_ref, cat_ref, inst_ref,
                     cat3_ref, inst3_ref, binf_ref, area_ref):
    f32 = jnp.float32
    cat3_ref[...] = jnp.zeros((4, _H, 128), f32)
    inst3_ref[...] = jnp.full((4, _H, 128), -1.0, f32)

    cw_iota = jax.lax.broadcasted_iota(jnp.int32, (1, _CWIN), 1)
    mcol = jax.lax.broadcasted_iota(jnp.int32, (_MW, _CWIN), 0)
    mrow = jax.lax.broadcasted_iota(jnp.int32, (_WIN, _MH), 1)
    win_iota = jax.lax.broadcasted_iota(jnp.int32, (_WIN, 1), 0).astype(f32)

    def window(j):
        y1 = sb_ref[0, j]
        x1 = sb_ref[1, j]
        r0 = jnp.minimum(jnp.maximum(y1.astype(jnp.int32) // 8, 0),
                         (_H - _WIN) // 8) * 8
        cb = jnp.minimum(jnp.maximum(x1.astype(jnp.int32) // 128, 0), 1)
        return r0, cb

    def paste(j):
        y1 = sb_ref[0, j]
        x1 = sb_ref[1, j]
        y2 = sb_ref[2, j]
        x2 = sb_ref[3, j]
        idx = sb_ref[6, j].astype(jnp.int32)
        h = jnp.maximum(y2 - y1, 1e-4)
        w = jnp.maximum(x2 - x1, 1e-4)
        r0, cb = window(j)

        xs = (cb * 128 + cw_iota).astype(f32) + 0.5
        u = (xs - x1) / w * _MW - 0.5
        u0 = jnp.floor(u)
        wu = u - u0
        u0i = u0.astype(jnp.int32)
        inside_x = ((xs >= x1) & (xs < x2)).astype(f32)
        wut = (jnp.where(mcol == u0i, 1.0 - wu, 0.0)
               + jnp.where(mcol == u0i + 1, wu, 0.0)) * inside_x

        ys = r0.astype(f32) + win_iota + 0.5
        v = (ys - y1) / h * _MH - 0.5
        v0 = jnp.floor(v)
        wv = v - v0
        v0i = v0.astype(jnp.int32)
        inside_y = ((ys >= y1) & (ys < y2)).astype(f32)
        wvm = (jnp.where(mrow == v0i, 1.0 - wv, 0.0)
               + jnp.where(mrow == v0i + 1, wv, 0.0)) * inside_y

        mask = masks_ref[idx]
        tmp = jax.lax.dot_general(
            mask, wut, (((1,), (0,)), ((), ())),
            precision=jax.lax.Precision.HIGHEST,
            preferred_element_type=f32)
        pm = jax.lax.dot_general(
            wvm, tmp, (((1,), (0,)), ((), ())),
            precision=jax.lax.Precision.HIGHEST,
            preferred_element_type=f32)

        binf = jnp.where(pm > _MASK_BIN, 1.0, 0.0)
        binf_ref[...] = binf
        area_ref[0] = jnp.sum(binf)

    def cond(i):
        s = sb_ref[4, jnp.minimum(i, _N - 1)]
        return (i < _N) & (s > _SCORE_T)

    def body(i):
        cls = sb_ref[5, i]
        idx = sb_ref[6, i].astype(jnp.int32)
        r0, cb = window(i)
        area = area_ref[0]
        binf = binf_ref[...]

        cat_w = [cat3_ref[cb + k, pl.ds(r0, _WIN), :] for k in range(3)]
        claimed = [c != 0.0 for c in cat_w]
        binf_k = [binf[:, 128 * k:128 * (k + 1)] for k in range(3)]
        ov = (jnp.sum(jnp.where(claimed[0], binf_k[0], 0.0))
              + jnp.sum(jnp.where(claimed[1], binf_k[1], 0.0))
              + jnp.sum(jnp.where(claimed[2], binf_k[2], 0.0)))
        ok = (area > 0.0) & (ov / jnp.maximum(area, 1.0) <= _OVERLAP_T)
        iv = (idx + 1).astype(f32)
        for k in range(3):
            new = (binf_k[k] != 0.0) & jnp.logical_not(claimed[k]) & ok
            cat3_ref[cb + k, pl.ds(r0, _WIN), :] = jnp.where(
                new, cls, cat_w[k])
            inst_w = inst3_ref[cb + k, pl.ds(r0, _WIN), :]
            inst3_ref[cb + k, pl.ds(r0, _WIN), :] = jnp.where(
                new, iv, inst_w)

        paste(jnp.minimum(i + 1, _N - 1))
        return i + 1

    pl.when(sb_ref[4, 0] > _SCORE_T)(lambda: paste(0))
    jax.lax.while_loop(cond, body, 0)

    seg3 = seg3_ref[...]
    cat3 = cat3_ref[...]
    free = cat3 == 0.0
    freeseg = jnp.where(free, seg3, 31)
    kbits = jnp.int32(0)
    for s in range(2, _NUM_SEM):
        cnt = jnp.sum((freeseg == s).astype(f32))
        kbits += jnp.where(cnt >= _STUFF_AREA, jnp.int32(1 << s), 0)
    hit = jnp.bitwise_and(jnp.right_shift(kbits, freeseg), 1) == 1
    cat_fin = jnp.where(hit, seg3.astype(f32) + float(_OFFSET), cat3)
    inst3 = inst3_ref[...]
    for c in range(4):
        cat_ref[:, 128 * c:128 * (c + 1)] = cat_fin[c]
        inst_ref[:, 128 * c:128 * (c + 1)] = inst3[c]


@functools.partial(jax.jit, static_argnums=())
def _run_single(boxes, scores, classes, masks, seg):
    order = jnp.argsort(-scores)
    bx = boxes[order]
    sb = jnp.stack([
        bx[:, 0], bx[:, 1], bx[:, 2], bx[:, 3],
        scores[order], classes[order].astype(jnp.float32),
        order.astype(jnp.float32), jnp.zeros((_N,), jnp.float32),
    ], axis=0)
    sb = jnp.pad(sb, ((0, 0), (0, 128 - _N)))
    seg3 = jnp.transpose(seg.reshape(_H, 4, 128), (1, 0, 2))

    cat, inst = pl.pallas_call(
        _panoptic_kernel,
        out_shape=(
            jax.ShapeDtypeStruct((_H, _W), jnp.float32),
            jax.ShapeDtypeStruct((_H, _W), jnp.float32),
        ),
        in_specs=[
            pl.BlockSpec(memory_space=pltpu.SMEM),
            pl.BlockSpec(memory_space=pltpu.VMEM),
            pl.BlockSpec(memory_space=pltpu.VMEM),
        ],
        out_specs=(
            pl.BlockSpec(memory_space=pltpu.VMEM),
            pl.BlockSpec(memory_space=pltpu.VMEM),
        ),
        scratch_shapes=[
            pltpu.VMEM((4, _H, 128), jnp.float32),
            pltpu.VMEM((4, _H, 128), jnp.float32),
            pltpu.VMEM((_WIN, _CWIN), jnp.float32),
            pltpu.SMEM((1,), jnp.float32),
        ],
    )(sb, masks, seg3)
    return cat, inst


def kernel(boxes, scores, classes, masks, segmentation_mask):
    B = boxes.shape[0]
    cats, insts = [], []
    for b in range(B):
        c, i = _run_single(boxes[b], scores[b], classes[b],
                           masks[b, :, :, :, 0], segmentation_mask[b])
        cats.append(c)
        insts.append(i)
    return jnp.stack(cats), jnp.stack(insts)
